# baseline (speedup 1.0000x reference)
import functools

import jax
import jax.numpy as jnp
from jax import lax
from jax.experimental import pallas as pl
from jax.experimental.pallas import tpu as pltpu
from jax.experimental.pallas import tpu_sc as plsc

_NC = 2
_NS = 16
_NW = _NC * _NS
_BB = 128
_L = 16


def kernel(inputs, word_emb_weight):
    B, T = inputs.shape
    V, D = word_emb_weight.shape
    n_bb = B // _BB
    assert n_bb == _NW and D == 32 and T % 2 == 0

    idx3 = inputs.T.reshape(T, n_bb, _BB).astype(jnp.int32)

    mesh = plsc.VectorSubcoreMesh(
        core_axis_name="c", subcore_axis_name="s",
        num_cores=_NC, num_subcores=_NS)

    @functools.partial(
        pl.kernel,
        out_type=jax.ShapeDtypeStruct((T, D // 8, n_bb, 8, _BB), jnp.float32),
        mesh=mesh,
        compiler_params=pltpu.CompilerParams(
            use_tc_tiling_on_sc=False, needs_layout_passes=False),
        scratch_types=[
            pltpu.VMEM((T, _BB), jnp.int32),
            pltpu.VMEM((2, _BB, D), jnp.float32),
            pltpu.VMEM((2, D, _BB), jnp.float32),
            pltpu.SemaphoreType.DMA,
            pltpu.SemaphoreType.DMA,
            pltpu.SemaphoreType.DMA,
            pltpu.SemaphoreType.DMA,
        ],
    )
    def emb(idx_hbm, table_hbm, out_hbm, idx_v, rows_v, tr_v, g0, g1, s0, s1):
        w = lax.axis_index("s") * _NC + lax.axis_index("c")
        pltpu.sync_copy(idx_hbm.at[:, w], idx_v)
        gs = (g0, g1)
        ss = (s0, s1)

        def gather(t, p):
            return pltpu.make_async_copy(
                table_hbm.at[idx_v.at[t]], rows_v.at[p], gs[p])

        def store(t, p, fb):
            return pltpu.make_async_copy(
                tr_v.at[p, pl.ds(fb * 8, 8)], out_hbm.at[t, fb, w], ss[p])

        lanes = lax.iota(jnp.int32, _L)

        def transpose(p):
            for f in range(D):
                col = jnp.full((_L,), f, jnp.int32)
                for blk in range(_BB // _L):
                    v = plsc.load_gather(
                        rows_v.at[p], [lanes + blk * _L, col])
                    tr_v[p, f, pl.ds(blk * _L, _L)] = v

        gather(0, 0).start()
        gather(1, 1).start()

        def body(half, carry):
            for p in range(2):
                t = half * 2 + p
                gather(t, p).wait()

                @pl.when(t >= 2)
                def _():
                    for fb in range(D // 8):
                        store(t - 2, p, fb).wait()

                transpose(p)

                @pl.when(t + 2 < T)
                def _():
                    gather(t + 2, p).start()

                for fb in range(D // 8):
                    store(t, p, fb).start()
            return carry

        lax.fori_loop(0, T // 2, body, 0)
        for p in range(2):
            for fb in range(D // 8):
                store(T - 2 + p, p, fb).wait()

    out5 = emb(idx3, word_emb_weight)
    out = out5.transpose(2, 4, 0, 1, 3).reshape(B, T, D)
    return out


# speedup vs baseline: 1.3593x; 1.3593x over previous
import functools

import jax
import jax.numpy as jnp
from jax import lax
from jax.experimental import pallas as pl
from jax.experimental.pallas import tpu as pltpu
from jax.experimental.pallas import tpu_sc as plsc

_NC = 2
_NS = 16
_NW = _NC * _NS
_BB = 128
_L = 16


def kernel(inputs, word_emb_weight):
    B, T = inputs.shape
    V, D = word_emb_weight.shape
    n_bb = B // _BB
    assert n_bb == _NW and D == 32 and T % 2 == 0

    idx3 = inputs.T.reshape(T, n_bb, _BB).astype(jnp.int32)

    mesh = plsc.VectorSubcoreMesh(
        core_axis_name="c", subcore_axis_name="s",
        num_cores=_NC, num_subcores=_NS)

    @functools.partial(
        pl.kernel,
        out_type=jax.ShapeDtypeStruct((T, D // 8, n_bb, 8, _BB), jnp.float32),
        mesh=mesh,
        compiler_params=pltpu.CompilerParams(
            use_tc_tiling_on_sc=False, needs_layout_passes=False),
        scratch_types=[
            pltpu.VMEM((T, _BB), jnp.int32),
            pltpu.VMEM((2, _BB, D), jnp.float32),
            pltpu.VMEM((2, D, _BB), jnp.float32),
            pltpu.SemaphoreType.DMA,
            pltpu.SemaphoreType.DMA,
            pltpu.SemaphoreType.DMA,
            pltpu.SemaphoreType.DMA,
        ],
    )
    def emb(idx_hbm, table_hbm, out_hbm, idx_v, rows_v, tr_v, g0, g1, s0, s1):
        w = lax.axis_index("s") * _NC + lax.axis_index("c")
        pltpu.sync_copy(idx_hbm.at[:, w], idx_v)
        gs = (g0, g1)
        ss = (s0, s1)

        def gather(t, p):
            return pltpu.make_async_copy(
                table_hbm.at[idx_v.at[t]], rows_v.at[p], gs[p])

        def store(t, p, fb):
            return pltpu.make_async_copy(
                tr_v.at[p, pl.ds(fb * 8, 8)], out_hbm.at[t, fb, w], ss[p])

        lanes = lax.iota(jnp.int32, _L)
        fcols = [jnp.bitwise_and(f + lanes, D - 1) for f in range(D)]
        rowvs = [lanes + blk * _L for blk in range(_BB // _L)]

        def transpose(p):
            for f in range(D):
                for blk in range(_BB // _L):
                    v = plsc.load_gather(rows_v.at[p], [rowvs[blk], fcols[f]])
                    plsc.store_scatter(tr_v.at[p], [fcols[f], rowvs[blk]], v)

        gather(0, 0).start()
        gather(1, 1).start()

        def body(half, carry):
            for p in range(2):
                t = half * 2 + p
                gather(t, p).wait()

                @pl.when(t >= 2)
                def _():
                    for fb in range(D // 8):
                        store(t - 2, p, fb).wait()

                transpose(p)

                @pl.when(t + 2 < T)
                def _():
                    gather(t + 2, p).start()

                for fb in range(D // 8):
                    store(t, p, fb).start()
            return carry

        lax.fori_loop(0, T // 2, body, 0)
        for p in range(2):
            for fb in range(D // 8):
                store(T - 2 + p, p, fb).wait()

    out5 = emb(idx3, word_emb_weight)
    out = out5.transpose(2, 4, 0, 1, 3).reshape(B, T, D)
    return out


# speedup vs baseline: 1.9677x; 1.4476x over previous
import functools

import jax
import jax.numpy as jnp
from jax import lax
from jax.experimental import pallas as pl
from jax.experimental.pallas import tpu as pltpu
from jax.experimental.pallas import tpu_sc as plsc

_NC = 2
_NS = 16
_NW = _NC * _NS
_BB = 128
_L = 16


def kernel(inputs, word_emb_weight):
    B, T = inputs.shape
    V, D = word_emb_weight.shape
    n_bb = B // _BB
    assert n_bb == _NW and D == 32 and T % 2 == 0

    idx3 = inputs.T.reshape(T, n_bb, _BB).astype(jnp.int32)

    mesh = plsc.VectorSubcoreMesh(
        core_axis_name="c", subcore_axis_name="s",
        num_cores=_NC, num_subcores=_NS)

    @functools.partial(
        pl.kernel,
        out_type=jax.ShapeDtypeStruct((T, D // 8, n_bb, 8, _BB), jnp.float32),
        mesh=mesh,
        compiler_params=pltpu.CompilerParams(
            use_tc_tiling_on_sc=False, needs_layout_passes=False,
            disable_bounds_checks=True),
        scratch_types=[
            pltpu.VMEM((T, _BB), jnp.int32),
            pltpu.VMEM((2, _BB, D), jnp.float32),
            pltpu.VMEM((2, D, _BB), jnp.float32),
            pltpu.SemaphoreType.DMA,
            pltpu.SemaphoreType.DMA,
            pltpu.SemaphoreType.DMA,
            pltpu.SemaphoreType.DMA,
        ],
    )
    def emb(idx_hbm, table_hbm, out_hbm, idx_v, rows_v, tr_v, g0, g1, s0, s1):
        w = lax.axis_index("s") * _NC + lax.axis_index("c")
        pltpu.sync_copy(idx_hbm.at[:, w], idx_v)
        gs = (g0, g1)
        ss = (s0, s1)

        def gather(t, p):
            return pltpu.make_async_copy(
                table_hbm.at[idx_v.at[t]], rows_v.at[p], gs[p])

        def store(t, p, fb):
            return pltpu.make_async_copy(
                tr_v.at[p, pl.ds(fb * 8, 8)], out_hbm.at[t, fb, w], ss[p])

        lanes = lax.iota(jnp.int32, _L)
        n_blk = _BB // _L

        def transpose(p):
            @functools.partial(
                plsc.parallel_loop, 0, D * n_blk, unroll=8)
            def _(i):
                f = i // n_blk
                blk = i % n_blk
                fcol = jnp.bitwise_and(lanes + f, D - 1)
                rowv = lanes + blk * _L
                v = plsc.load_gather(rows_v.at[p], [rowv, fcol])
                plsc.store_scatter(tr_v.at[p], [fcol, rowv], v)

        gather(0, 0).start()
        gather(1, 1).start()

        def body(half, carry):
            for p in range(2):
                t = half * 2 + p
                gather(t, p).wait()

                @pl.when(t >= 2)
                def _():
                    for fb in range(D // 8):
                        store(t - 2, p, fb).wait()

                transpose(p)

                @pl.when(t + 2 < T)
                def _():
                    gather(t + 2, p).start()

                for fb in range(D // 8):
                    store(t, p, fb).start()
            return carry

        lax.fori_loop(0, T // 2, body, 0)
        for p in range(2):
            for fb in range(D // 8):
                store(T - 2 + p, p, fb).wait()

    out5 = emb(idx3, word_emb_weight)
    out = out5.transpose(2, 4, 0, 1, 3).reshape(B, T, D)
    return out
